# baseline (speedup 1.0000x reference)
import jax
import jax.numpy as jnp
from jax.experimental import pallas as pl

NP_ = 576
PD_ = 768


def _add_kernel(x_ref, t_ref, o_ref):
    o_ref[...] = x_ref[...] + t_ref[...]


def kernel(encoded_patches, pos_table):
    b = encoded_patches.shape[0]
    return pl.pallas_call(
        _add_kernel,
        grid=(b,),
        in_specs=[
            pl.BlockSpec((1, NP_, PD_), lambda i: (i, 0, 0)),
            pl.BlockSpec((NP_, PD_), lambda i: (0, 0)),
        ],
        out_specs=pl.BlockSpec((1, NP_, PD_), lambda i: (i, 0, 0)),
        out_shape=jax.ShapeDtypeStruct(encoded_patches.shape, encoded_patches.dtype),
    )(encoded_patches, pos_table)


# speedup vs baseline: 1.1943x; 1.1943x over previous
import jax
import jax.numpy as jnp
from jax.experimental import pallas as pl

NP_ = 576
PD_ = 768


def _add_kernel(x_ref, t_ref, o_ref):
    o_ref[...] = x_ref[...] + t_ref[...]


BB_ = 4


def kernel(encoded_patches, pos_table):
    b = encoded_patches.shape[0]
    return pl.pallas_call(
        _add_kernel,
        grid=(b // BB_,),
        in_specs=[
            pl.BlockSpec((BB_, NP_, PD_), lambda i: (i, 0, 0)),
            pl.BlockSpec((NP_, PD_), lambda i: (0, 0)),
        ],
        out_specs=pl.BlockSpec((BB_, NP_, PD_), lambda i: (i, 0, 0)),
        out_shape=jax.ShapeDtypeStruct(encoded_patches.shape, encoded_patches.dtype),
    )(encoded_patches, pos_table)


# speedup vs baseline: 1.2104x; 1.0135x over previous
import jax
import jax.numpy as jnp
from jax.experimental import pallas as pl

NP_ = 576
PD_ = 768


def _add_kernel(x_ref, t_ref, o_ref):
    o_ref[...] = x_ref[...] + t_ref[...]


BB_ = 8


def kernel(encoded_patches, pos_table):
    b = encoded_patches.shape[0]
    return pl.pallas_call(
        _add_kernel,
        grid=(b // BB_,),
        in_specs=[
            pl.BlockSpec((BB_, NP_, PD_), lambda i: (i, 0, 0)),
            pl.BlockSpec((NP_, PD_), lambda i: (0, 0)),
        ],
        out_specs=pl.BlockSpec((BB_, NP_, PD_), lambda i: (i, 0, 0)),
        out_shape=jax.ShapeDtypeStruct(encoded_patches.shape, encoded_patches.dtype),
    )(encoded_patches, pos_table)
